# baseline (speedup 1.0000x reference)
import functools

import jax
import jax.numpy as jnp
from jax import lax
from jax.experimental import pallas as pl
from jax.experimental.pallas import tpu as pltpu
from jax.experimental.pallas import tpu_sc as plsc

_INFO = plsc.get_sparse_core_info()
_NC = _INFO.num_cores
_NS = _INFO.num_subcores
_NW = _NC * _NS
_L = _INFO.num_lanes

_B = 16384
_J = 16
_K = 64
_NI = 100000
_BPW = _B // _NW
_EPW = _BPW * _J
_ROWS = _EPW // 128
_NG = _BPW // _L


def _sc_gather(xflat, idx0, idx2):
    mesh = plsc.VectorSubcoreMesh(core_axis_name="c", subcore_axis_name="s")

    @functools.partial(
        pl.kernel,
        out_type=jax.ShapeDtypeStruct((_B * _J,), jnp.float32),
        mesh=mesh,
        compiler_params=pltpu.CompilerParams(needs_layout_passes=False),
        scratch_types=[
            pltpu.VMEM((_BPW,), jnp.int32),
            pltpu.VMEM((_BPW,), jnp.int32),
            pltpu.VMEM((_EPW,), jnp.int32),
            pltpu.VMEM((_EPW,), jnp.float32),
            pltpu.SemaphoreType.DMA,
        ],
    )
    def k(x_hbm, idx0_hbm, idx2_hbm, out_hbm, i0_v, i2_v, idx_v, out_v, sem):
        wid = lax.axis_index("s") * _NC + lax.axis_index("c")
        base_b = wid * _BPW
        pltpu.sync_copy(idx0_hbm.at[pl.ds(base_b, _BPW)], i0_v)
        pltpu.sync_copy(idx2_hbm.at[pl.ds(base_b, _BPW)], i2_v)

        lane = lax.iota(jnp.int32, _L)

        def gbody(g, carry):
            i0 = i0_v[pl.ds(g * _L, _L)]
            i2 = i2_v[pl.ds(g * _L, _L)]
            base2 = i2 * _NI + i0
            pb = g * 256 + lane * _J
            for j in range(_J):
                plsc.store_scatter(idx_v, [pb + j], base2 + j * (_K * _NI))
            pltpu.async_copy(
                x_hbm.at[idx_v.at[pl.ds(g * 256, 128)]],
                out_v.at[pl.ds(g * 256, 128)], sem)
            pltpu.async_copy(
                x_hbm.at[idx_v.at[pl.ds(g * 256 + 128, 128)]],
                out_v.at[pl.ds(g * 256 + 128, 128)], sem)
            return carry

        lax.fori_loop(0, _NG, gbody, 0)
        pltpu.make_async_copy(x_hbm.at[pl.ds(0, _EPW)], out_v, sem).wait()
        pltpu.sync_copy(out_v, out_hbm.at[pl.ds(base_b * _J, _EPW)])

    return k(xflat, idx0, idx2)


def kernel(x, idx0, idx2):
    from jax.experimental.layout import Format, Layout
    from jax.sharding import SingleDeviceSharding
    xt = x.transpose(1, 2, 0)
    fmt = Format(Layout(major_to_minor=(0, 1, 2), tiling=((8,),)),
                 SingleDeviceSharding(jax.devices()[0]))
    xt8 = jax.device_put(xt, fmt)
    xflat = xt8.reshape(-1)
    out = _sc_gather(xflat, idx0.astype(jnp.int32), idx2.astype(jnp.int32))
    return out.reshape(_B, _J)
